# Initial kernel scaffold; baseline (speedup 1.0000x reference)
import jax
import jax.numpy as jnp
from jax.experimental import pallas as pl


def kernel(x1, x2, gamma, scales1, zero_points1):
    raise NotImplementedError("write your pallas kernel here")


# speedup vs baseline: 1.4611x; 1.4611x over previous
import jax
import jax.numpy as jnp
from jax.experimental import pallas as pl
from jax.experimental.pallas import tpu as pltpu

_EPS = 1e-06
_QMIN, _QMAX = -128.0, 127.0


def _fused_body(x1_ref, x2_ref, g_ref, s_ref, zp_ref, o_ref):
    x = x1_ref[...] + x2_ref[...]
    ms = jnp.mean(x * x, axis=-1, keepdims=True)
    inv = jax.lax.rsqrt(ms + _EPS)
    mult = g_ref[...] / s_ref[...]
    y = x * (inv * mult)
    q = jnp.clip(jnp.round(y) + zp_ref[...], _QMIN, _QMAX)
    o_ref[...] = q.astype(jnp.int8)


def kernel(x1, x2, gamma, scales1, zero_points1):
    B, S, D = x1.shape
    rows = B * S
    x1f = x1.reshape(rows, D)
    x2f = x2.reshape(rows, D)
    g = gamma.reshape(1, D)
    s = scales1.reshape(1, D)
    zp = zero_points1.reshape(1, D)

    BR = 256
    grid = (rows // BR,)

    row_spec = pl.BlockSpec((BR, D), lambda i: (i, 0))
    vec_spec = pl.BlockSpec((1, D), lambda i: (0, 0))

    out = pl.pallas_call(
        _fused_body,
        out_shape=jax.ShapeDtypeStruct((rows, D), jnp.int8),
        grid=grid,
        in_specs=[row_spec, row_spec, vec_spec, vec_spec, vec_spec],
        out_specs=row_spec,
        compiler_params=pltpu.CompilerParams(
            dimension_semantics=("arbitrary",),
        ),
        name="fused_rmsnorm_quant",
    )(x1f, x2f, g, s, zp)
    return out.reshape(B, S, D)


# speedup vs baseline: 1.6877x; 1.1551x over previous
import jax
import jax.numpy as jnp
from jax.experimental import pallas as pl
from jax.experimental.pallas import tpu as pltpu

_EPS = 1e-06
_QMIN, _QMAX = -128.0, 127.0


def _fused_body(x1_ref, x2_ref, g_ref, s_ref, zp_ref, o_ref):
    x = x1_ref[...] + x2_ref[...]
    ms = jnp.mean(x * x, axis=-1, keepdims=True)
    inv = jax.lax.rsqrt(ms + _EPS)
    mult = g_ref[...] / s_ref[...]
    y = x * (inv * mult)
    q = jnp.clip(jnp.round(y) + zp_ref[...], _QMIN, _QMAX)
    o_ref[...] = q.astype(jnp.int8)


def kernel(x1, x2, gamma, scales1, zero_points1):
    B, S, D = x1.shape
    rows = B * S
    x1f = x1.reshape(rows, D)
    x2f = x2.reshape(rows, D)
    g = gamma.reshape(1, D)
    s = scales1.reshape(1, D)
    zp = zero_points1.reshape(1, D)

    BR = 512
    grid = (rows // BR,)

    row_spec = pl.BlockSpec((BR, D), lambda i: (i, 0))
    vec_spec = pl.BlockSpec((1, D), lambda i: (0, 0))

    out = pl.pallas_call(
        _fused_body,
        out_shape=jax.ShapeDtypeStruct((rows, D), jnp.int8),
        grid=grid,
        in_specs=[row_spec, row_spec, vec_spec, vec_spec, vec_spec],
        out_specs=row_spec,
        compiler_params=pltpu.CompilerParams(
            dimension_semantics=("arbitrary",),
        ),
        name="fused_rmsnorm_quant",
    )(x1f, x2f, g, s, zp)
    return out.reshape(B, S, D)


# speedup vs baseline: 1.7762x; 1.0524x over previous
import jax
import jax.numpy as jnp
from jax.experimental import pallas as pl
from jax.experimental.pallas import tpu as pltpu

_EPS = 1e-06
_QMIN, _QMAX = -128.0, 127.0


def _fused_body(x1_ref, x2_ref, g_ref, s_ref, zp_ref, o_ref):
    x = x1_ref[...] + x2_ref[...]
    ms = jnp.mean(x * x, axis=-1, keepdims=True)
    inv = jax.lax.rsqrt(ms + _EPS)
    mult = g_ref[...] / s_ref[...]
    y = x * (inv * mult)
    q = jnp.clip(jnp.round(y) + zp_ref[...], _QMIN, _QMAX)
    o_ref[...] = q.astype(jnp.int8)


def kernel(x1, x2, gamma, scales1, zero_points1):
    B, S, D = x1.shape
    rows = B * S
    x1f = x1.reshape(rows, D)
    x2f = x2.reshape(rows, D)
    g = gamma.reshape(1, D)
    s = scales1.reshape(1, D)
    zp = zero_points1.reshape(1, D)

    BR = 1024
    grid = (rows // BR,)

    row_spec = pl.BlockSpec((BR, D), lambda i: (i, 0))
    vec_spec = pl.BlockSpec((1, D), lambda i: (0, 0))

    out = pl.pallas_call(
        _fused_body,
        out_shape=jax.ShapeDtypeStruct((rows, D), jnp.int8),
        grid=grid,
        in_specs=[row_spec, row_spec, vec_spec, vec_spec, vec_spec],
        out_specs=row_spec,
        compiler_params=pltpu.CompilerParams(
            dimension_semantics=("arbitrary",),
        ),
        name="fused_rmsnorm_quant",
    )(x1f, x2f, g, s, zp)
    return out.reshape(B, S, D)
